# baseline (speedup 1.0000x reference)
import functools

import jax
import jax.numpy as jnp
from jax import lax
from jax.experimental import pallas as pl
from jax.experimental.pallas import tpu as pltpu
from jax.experimental.pallas import tpu_sc as plsc

_L = 16


def _lane_allsum(v):
    iota = lax.iota(jnp.int32, _L)
    dnums = lax.GatherDimensionNumbers(
        offset_dims=(), collapsed_slice_dims=(0,), start_index_map=(0,))
    for sh in (1, 2, 4, 8):
        idx = lax.bitwise_xor(iota, jnp.int32(sh))
        v = v + lax.gather(v, idx[:, None], dnums, slice_sizes=(1,),
                           mode=lax.GatherScatterMode.PROMISE_IN_BOUNDS)
    return v


@functools.cache
def _kappa_sc(n: int):
    n_full = n // _L
    rem = n - n_full * _L
    mesh = plsc.VectorSubcoreMesh(
        core_axis_name="c", subcore_axis_name="s", num_cores=1)

    @functools.partial(
        pl.kernel,
        mesh=mesh,
        out_type=jax.ShapeDtypeStruct((1,), jnp.float32),
        scratch_types=[
            pltpu.VMEM((n,), jnp.int32),
            pltpu.VMEM((n,), jnp.int32),
            pltpu.VMEM((_L,), jnp.float32),
            pltpu.SemaphoreType.DMA,
            pltpu.SemaphoreType.DMA,
        ],
    )
    def kern(pred_hbm, gt_hbm, out_hbm, pred_v, gt_v, out_v, sem_p, sem_g):
        wid = lax.axis_index("s")

        @pl.when(wid == 0)
        def _():
            cp_p = pltpu.async_copy(pred_hbm, pred_v, sem_p)
            cp_g = pltpu.async_copy(gt_hbm, gt_v, sem_g)
            cp_p.wait()
            cp_g.wait()
            z = jnp.zeros((_L,), jnp.int32)
            accd, s1g, s1p, s2g, s2p = z, z, z, z, z

            def step(g, p, acc):
                accd, s1g, s1p, s2g, s2p = acc
                d = g - p
                return (accd + d * d, s1g + g, s1p + p,
                        s2g + g * g, s2p + p * p)

            unroll = 8
            n_loop = n_full // unroll

            def body(i, acc):
                base = i * (_L * unroll)
                for j in range(unroll):
                    g = gt_v[pl.ds(base + j * _L, _L)]
                    p = pred_v[pl.ds(base + j * _L, _L)]
                    acc = step(g, p, acc)
                return acc

            acc = lax.fori_loop(
                0, n_loop, body, (accd, s1g, s1p, s2g, s2p))
            for j in range(n_loop * unroll, n_full):
                g = gt_v[pl.ds(j * _L, _L)]
                p = pred_v[pl.ds(j * _L, _L)]
                acc = step(g, p, acc)
            accd, s1g, s1p, s2g, s2p = acc
            if rem:
                mask = lax.iota(jnp.int32, _L) >= jnp.int32(_L - rem)
                g = jnp.where(mask, gt_v[pl.ds(n - _L, _L)], 0)
                p = jnp.where(mask, pred_v[pl.ds(n - _L, _L)], 0)
                accd, s1g, s1p, s2g, s2p = step(
                    g, p, (accd, s1g, s1p, s2g, s2p))

            vd = _lane_allsum(accd).astype(jnp.float32)
            v1g = _lane_allsum(s1g).astype(jnp.float32)
            v1p = _lane_allsum(s1p).astype(jnp.float32)
            v2g = _lane_allsum(s2g).astype(jnp.float32)
            v2p = _lane_allsum(s2p).astype(jnp.float32)
            nf = jnp.float32(n)
            den = nf * (v2g + v2p) - 2.0 * v1g * v1p
            res = 1.0 - nf * vd / den
            out_v[...] = res
            pltpu.sync_copy(out_v.at[pl.ds(0, 1)], out_hbm)

    return kern


def kernel(y_pred, y_gt):
    y_pred = jnp.ravel(y_pred).astype(jnp.int32)
    y_gt = jnp.ravel(y_gt).astype(jnp.int32)
    n = y_gt.shape[0]
    out = _kappa_sc(n)(y_pred, y_gt)
    return jnp.reshape(out, ())


# speedup vs baseline: 1.0067x; 1.0067x over previous
import functools

import jax
import jax.numpy as jnp
from jax import lax
from jax.experimental import pallas as pl
from jax.experimental.pallas import tpu as pltpu
from jax.experimental.pallas import tpu_sc as plsc

_L = 16


def _lane_allsum(v):
    iota = lax.iota(jnp.int32, _L)
    dnums = lax.GatherDimensionNumbers(
        offset_dims=(), collapsed_slice_dims=(0,), start_index_map=(0,))
    for sh in (1, 2, 4, 8):
        idx = lax.bitwise_xor(iota, jnp.int32(sh))
        v = v + lax.gather(v, idx[:, None], dnums, slice_sizes=(1,),
                           mode=lax.GatherScatterMode.PROMISE_IN_BOUNDS)
    return v


@functools.cache
def _kappa_sc(n: int):
    n_full = n // _L
    rem = n - n_full * _L
    mesh = plsc.VectorSubcoreMesh(
        core_axis_name="c", subcore_axis_name="s", num_cores=1)

    @functools.partial(
        pl.kernel,
        mesh=mesh,
        out_type=jax.ShapeDtypeStruct((1,), jnp.float32),
        scratch_types=[
            pltpu.VMEM((n,), jnp.int32),
            pltpu.VMEM((n,), jnp.int32),
            pltpu.VMEM((_L,), jnp.float32),
            pltpu.SemaphoreType.DMA,
            pltpu.SemaphoreType.DMA,
        ],
    )
    def kern(pred_hbm, gt_hbm, out_hbm, pred_v, gt_v, out_v, sem_p, sem_g):
        wid = lax.axis_index("s")

        @pl.when(wid == 0)
        def _():
            cp_p = pltpu.async_copy(pred_hbm, pred_v, sem_p)
            cp_g = pltpu.async_copy(gt_hbm, gt_v, sem_g)
            cp_p.wait()
            cp_g.wait()
            z = jnp.zeros((_L,), jnp.int32)
            accd, s1g, s1p, s2g, s2p = z, z, z, z, z

            def step(g, p, acc):
                accd, s1g, s1p, s2g, s2p = acc
                d = g - p
                return (accd + d * d, s1g + g, s1p + p,
                        s2g + g * g, s2p + p * p)

            unroll = 4
            n_loop = n_full // unroll

            def body(i, acc):
                base = i * (_L * unroll)
                for j in range(unroll):
                    g = gt_v[pl.ds(base + j * _L, _L)]
                    p = pred_v[pl.ds(base + j * _L, _L)]
                    acc = step(g, p, acc)
                return acc

            acc = lax.fori_loop(
                0, n_loop, body, (accd, s1g, s1p, s2g, s2p))
            for j in range(n_loop * unroll, n_full):
                g = gt_v[pl.ds(j * _L, _L)]
                p = pred_v[pl.ds(j * _L, _L)]
                acc = step(g, p, acc)
            accd, s1g, s1p, s2g, s2p = acc
            if rem:
                mask = lax.iota(jnp.int32, _L) >= jnp.int32(_L - rem)
                g = jnp.where(mask, gt_v[pl.ds(n - _L, _L)], 0)
                p = jnp.where(mask, pred_v[pl.ds(n - _L, _L)], 0)
                accd, s1g, s1p, s2g, s2p = step(
                    g, p, (accd, s1g, s1p, s2g, s2p))

            vd = _lane_allsum(accd).astype(jnp.float32)
            v1g = _lane_allsum(s1g).astype(jnp.float32)
            v1p = _lane_allsum(s1p).astype(jnp.float32)
            v2g = _lane_allsum(s2g).astype(jnp.float32)
            v2p = _lane_allsum(s2p).astype(jnp.float32)
            nf = jnp.float32(n)
            den = nf * (v2g + v2p) - 2.0 * v1g * v1p
            res = 1.0 - nf * vd / den
            out_v[...] = res
            pltpu.sync_copy(out_v.at[pl.ds(0, 1)], out_hbm)

    return kern


def kernel(y_pred, y_gt):
    y_pred = jnp.ravel(y_pred).astype(jnp.int32)
    y_gt = jnp.ravel(y_gt).astype(jnp.int32)
    n = y_gt.shape[0]
    out = _kappa_sc(n)(y_pred, y_gt)
    return jnp.reshape(out, ())
